# Initial kernel scaffold; baseline (speedup 1.0000x reference)
import jax
import jax.numpy as jnp
from jax.experimental import pallas as pl


def kernel(feat, loc, W, b):
    raise NotImplementedError("write your pallas kernel here")


# speedup vs baseline: 1.1926x; 1.1926x over previous
import jax
import jax.numpy as jnp
from jax.experimental import pallas as pl


def _transpose_block(feat_ref, out_ref):
    out_ref[...] = feat_ref[...].T


def kernel(feat, loc, W, b):
    del loc, W, b
    bsz, c, n = feat.shape
    tn = 512
    grid = (bsz, n // tn)
    return pl.pallas_call(
        _transpose_block,
        grid=grid,
        in_specs=[pl.BlockSpec((None, c, tn), lambda i, j: (i, 0, j))],
        out_specs=pl.BlockSpec((None, tn, c), lambda i, j: (i, j, 0)),
        out_shape=jax.ShapeDtypeStruct((bsz, n, c), feat.dtype),
    )(feat)


# speedup vs baseline: 2.1742x; 1.8231x over previous
import jax
import jax.numpy as jnp
from jax.experimental import pallas as pl


def _transpose_block(feat_ref, out_ref):
    out_ref[...] = feat_ref[...].T


def kernel(feat, loc, W, b):
    del loc, W, b
    bsz, c, n = feat.shape
    tn = 2048
    grid = (bsz, n // tn)
    return pl.pallas_call(
        _transpose_block,
        grid=grid,
        in_specs=[pl.BlockSpec((None, c, tn), lambda i, j: (i, 0, j))],
        out_specs=pl.BlockSpec((None, tn, c), lambda i, j: (i, j, 0)),
        out_shape=jax.ShapeDtypeStruct((bsz, n, c), feat.dtype),
    )(feat)


# speedup vs baseline: 2.3616x; 1.0862x over previous
import jax
import jax.numpy as jnp
from jax.experimental import pallas as pl


def _transpose_block(feat_ref, out_ref):
    out_ref[...] = feat_ref[...].T


def kernel(feat, loc, W, b):
    del loc, W, b
    bsz, c, n = feat.shape
    tn = 4096
    grid = (bsz, n // tn)
    return pl.pallas_call(
        _transpose_block,
        grid=grid,
        in_specs=[pl.BlockSpec((None, c, tn), lambda i, j: (i, 0, j))],
        out_specs=pl.BlockSpec((None, tn, c), lambda i, j: (i, j, 0)),
        out_shape=jax.ShapeDtypeStruct((bsz, n, c), feat.dtype),
    )(feat)


# speedup vs baseline: 2.6934x; 1.1405x over previous
import jax
import jax.numpy as jnp
from jax.experimental import pallas as pl


def _transpose_block(feat_ref, out_ref):
    out_ref[...] = jnp.swapaxes(feat_ref[...], 1, 2)


def kernel(feat, loc, W, b):
    del loc, W, b
    bsz, c, n = feat.shape
    tb = 2
    grid = (bsz // tb,)
    return pl.pallas_call(
        _transpose_block,
        grid=grid,
        in_specs=[pl.BlockSpec((tb, c, n), lambda i: (i, 0, 0))],
        out_specs=pl.BlockSpec((tb, n, c), lambda i: (i, 0, 0)),
        out_shape=jax.ShapeDtypeStruct((bsz, n, c), feat.dtype),
    )(feat)
